# baseline (speedup 1.0000x reference)
import functools

import jax
import jax.numpy as jnp
from jax import lax
from jax.experimental import pallas as pl
from jax.experimental.pallas import tpu as pltpu
from jax.experimental.pallas import tpu_sc as plsc

NUM_EMB = 1000000
DIM = 32
BATCH = 16384
HIST = 50
TOTAL = BATCH * HIST

IDX_W = 32
N_ROWS = TOTAL // IDX_W
JB = 128 // IDX_W
GJB = 10
K = GJB * JB


def _gather_sc(flat_idx2d, weight):
    info = plsc.get_sparse_core_info()
    nw = info.num_cores * info.num_subcores
    rows_per_w = N_ROWS // nw
    jb_per_w = rows_per_w // JB
    groups = jb_per_w // GJB
    mesh = plsc.VectorSubcoreMesh(core_axis_name="c", subcore_axis_name="s")

    @functools.partial(
        pl.kernel,
        out_type=jax.ShapeDtypeStruct((HIST, DIM // 8, BATCH // 128, 8, 128),
                                      jnp.float32),
        mesh=mesh,
        scratch_types=[
            pltpu.VMEM((rows_per_w, IDX_W), jnp.int32),
            pltpu.VMEM((GJB, 128, DIM), jnp.float32),
            pltpu.VMEM((GJB, 128, DIM), jnp.float32),
            pltpu.VMEM((2, DIM // 8, 8, 128), jnp.float32),
            pltpu.SemaphoreType.DMA,
            pltpu.SemaphoreType.DMA,
            pltpu.SemaphoreType.DMA,
            pltpu.SemaphoreType.DMA,
        ],
        compiler_params=pltpu.CompilerParams(use_tc_tiling_on_sc=False,
                                             needs_layout_passes=False),
    )
    def k(idx_hbm, table_hbm, out_hbm, idx_v, rows0, rows1, t_v, gs0, gs1,
          ws0, ws1):
        wid = lax.axis_index("s") * info.num_cores + lax.axis_index("c")
        base = wid * rows_per_w
        jb_base = wid * jb_per_w
        rows_v = (rows0, rows1)
        gsem = (gs0, gs1)
        wsem = (ws0, ws1)
        lane = lax.iota(jnp.int32, 16)

        pltpu.sync_copy(idx_hbm.at[pl.ds(base, rows_per_w)], idx_v)

        def fire_gather(g, b):
            for t in range(GJB):
                for q in range(JB):
                    r = (g * GJB + t) * JB + q
                    pltpu.async_copy(table_hbm.at[idx_v.at[r]],
                                     rows_v[b].at[t, pl.ds(IDX_W * q, IDX_W)],
                                     gsem[b])

        def wait_gather(g, b):
            for t in range(GJB):
                for q in range(JB):
                    r = (g * GJB + t) * JB + q
                    pltpu.make_async_copy(
                        table_hbm.at[idx_v.at[r]],
                        rows_v[b].at[t, pl.ds(IDX_W * q, IDX_W)],
                        gsem[b]).wait()

        def out_descr(s, jb):
            h = jb // (BATCH // 128)
            bb = lax.rem(jb, BATCH // 128)
            return [(t_v.at[s, cb], out_hbm.at[h, cb, bb])
                    for cb in range(DIM // 8)]

        def fire_out(s, jb):
            for src, dst in out_descr(s, jb):
                pltpu.async_copy(src, dst, wsem[s])

        def wait_out(s, jb):
            for src, dst in out_descr(s, jb):
                pltpu.make_async_copy(src, dst, wsem[s]).wait()

        def transpose_block(b, t, s):
            src = rows_v[b].at[t]
            for cb in range(DIM // 8):
                for ci in range(8):
                    c = cb * 8 + ci
                    cvec = jnp.full((16,), c, jnp.int32)
                    for l0 in range(0, 128, 16):
                        v = plsc.load_gather(src, [lane + l0, cvec])
                        t_v[s, cb, ci, pl.ds(l0, 16)] = v

        def process_group(g, b):
            @pl.loop(0, GJB // 2)
            def _(t2):
                pair = (g * GJB) // 2 + t2
                for s in range(2):
                    t = t2 * 2 + s
                    jb = jb_base + g * GJB + t

                    @pl.when(pair > 0)
                    def _():
                        wait_out(s, jb - 2)
                    transpose_block(b, t, s)
                    fire_out(s, jb)

        fire_gather(0, 0)

        @pl.loop(0, groups, step=2)
        def _(g):
            fire_gather(g + 1, 1)
            wait_gather(g, 0)
            process_group(g, 0)

            @pl.when(g + 2 < groups)
            def _():
                fire_gather(g + 2, 0)
            wait_gather(g + 1, 1)
            process_group(g + 1, 1)

        last = jb_base + jb_per_w
        wait_out(0, last - 2)
        wait_out(1, last - 1)

    return k(flat_idx2d, weight)


_gather_jit = jax.jit(_gather_sc)


def kernel(indices, weight):
    flat = indices.T.reshape(N_ROWS, IDX_W)
    out5 = _gather_jit(flat, weight)
    return (out5.transpose(2, 4, 0, 1, 3)
            .reshape(BATCH, HIST, DIM))


# speedup vs baseline: 1.8742x; 1.8742x over previous
import functools

import jax
import jax.numpy as jnp
from jax import lax
from jax.experimental import pallas as pl
from jax.experimental.pallas import tpu as pltpu
from jax.experimental.pallas import tpu_sc as plsc

NUM_EMB = 1000000
DIM = 32
BATCH = 16384
HIST = 50
TOTAL = BATCH * HIST

IDX_W = 32
N_ROWS = TOTAL // IDX_W
JB = 128 // IDX_W
GJB = 10
K = GJB * JB


def _gather_sc(flat_idx2d, weight):
    info = plsc.get_sparse_core_info()
    nw = info.num_cores * info.num_subcores
    rows_per_w = N_ROWS // nw
    jb_per_w = rows_per_w // JB
    groups = jb_per_w // GJB
    mesh = plsc.VectorSubcoreMesh(core_axis_name="c", subcore_axis_name="s")

    @functools.partial(
        pl.kernel,
        out_type=jax.ShapeDtypeStruct((HIST, DIM // 8, BATCH // 128, 8, 128),
                                      jnp.float32),
        mesh=mesh,
        scratch_types=[
            pltpu.VMEM((rows_per_w, IDX_W), jnp.int32),
            pltpu.VMEM((GJB, 128, DIM), jnp.float32),
            pltpu.VMEM((GJB, 128, DIM), jnp.float32),
            pltpu.VMEM((2, DIM, 129), jnp.float32),
            pltpu.SemaphoreType.DMA,
            pltpu.SemaphoreType.DMA,
            pltpu.SemaphoreType.DMA,
            pltpu.SemaphoreType.DMA,
        ],
        compiler_params=pltpu.CompilerParams(use_tc_tiling_on_sc=False,
                                             needs_layout_passes=False),
    )
    def k(idx_hbm, table_hbm, out_hbm, idx_v, rows0, rows1, t_v, gs0, gs1,
          ws0, ws1):
        wid = lax.axis_index("s") * info.num_cores + lax.axis_index("c")
        base = wid * rows_per_w
        jb_base = wid * jb_per_w
        rows_v = (rows0, rows1)
        gsem = (gs0, gs1)
        wsem = (ws0, ws1)
        lane = lax.iota(jnp.int32, 16)

        pltpu.sync_copy(idx_hbm.at[pl.ds(base, rows_per_w)], idx_v)

        def fire_gather(g, b):
            for t in range(GJB):
                for q in range(JB):
                    r = (g * GJB + t) * JB + q
                    pltpu.async_copy(table_hbm.at[idx_v.at[r]],
                                     rows_v[b].at[t, pl.ds(IDX_W * q, IDX_W)],
                                     gsem[b])

        def wait_gather(g, b):
            for t in range(GJB):
                for q in range(JB):
                    r = (g * GJB + t) * JB + q
                    pltpu.make_async_copy(
                        table_hbm.at[idx_v.at[r]],
                        rows_v[b].at[t, pl.ds(IDX_W * q, IDX_W)],
                        gsem[b]).wait()

        def out_descr(s, jb):
            h = jb // (BATCH // 128)
            bb = lax.rem(jb, BATCH // 128)
            return [(t_v.at[s, pl.ds(8 * cb, 8), pl.ds(0, 128)],
                     out_hbm.at[h, cb, bb]) for cb in range(DIM // 8)]

        def fire_out(s, jb):
            for src, dst in out_descr(s, jb):
                pltpu.async_copy(src, dst, wsem[s])

        def wait_out(s, jb):
            for src, dst in out_descr(s, jb):
                pltpu.make_async_copy(src, dst, wsem[s]).wait()

        def transpose_block(b, t, s):
            src = rows_v[b].at[t]
            dst = t_v.at[s]

            @pl.loop(0, 8)
            def _(l8):
                lbase = jnp.broadcast_to(l8 * 16, (16,)).astype(jnp.int32)
                for li in range(16):
                    l = l8 * 16 + li
                    lvec = lbase + li
                    v0 = src[l, pl.ds(0, 16)]
                    v1 = src[l, pl.ds(16, 16)]
                    plsc.store_scatter(dst, [lane, lvec], v0)
                    plsc.store_scatter(dst, [lane + 16, lvec], v1)

        def process_group(g, b):
            @pl.loop(0, GJB // 2)
            def _(t2):
                pair = (g * GJB) // 2 + t2
                for s in range(2):
                    t = t2 * 2 + s
                    jb = jb_base + g * GJB + t

                    @pl.when(pair > 0)
                    def _():
                        wait_out(s, jb - 2)
                    transpose_block(b, t, s)
                    fire_out(s, jb)

        fire_gather(0, 0)

        @pl.loop(0, groups, step=2)
        def _(g):
            fire_gather(g + 1, 1)
            wait_gather(g, 0)
            process_group(g, 0)

            @pl.when(g + 2 < groups)
            def _():
                fire_gather(g + 2, 0)
            wait_gather(g + 1, 1)
            process_group(g + 1, 1)

        last = jb_base + jb_per_w
        wait_out(0, last - 2)
        wait_out(1, last - 1)

    return k(flat_idx2d, weight)


_gather_jit = jax.jit(_gather_sc)


def kernel(indices, weight):
    flat = indices.T.reshape(N_ROWS, IDX_W)
    out5 = _gather_jit(flat, weight)
    return (out5.transpose(2, 4, 0, 1, 3)
            .reshape(BATCH, HIST, DIM))
